# Initial kernel scaffold; baseline (speedup 1.0000x reference)
import jax
import jax.numpy as jnp
from jax.experimental import pallas as pl


def kernel(init_embed, init_rel, w_rel, W_in, b_in, V0, a0, root0, bias0, V1, a1, root1, bias1, edge_index, edge_type):
    raise NotImplementedError("write your pallas kernel here")


# speedup vs baseline: 1.8053x; 1.8053x over previous
import functools

import jax
import jax.numpy as jnp
from jax import lax
from jax.experimental import pallas as pl
from jax.experimental.pallas import tpu as pltpu
from jax.experimental.pallas import tpu_sc as plsc

N_ENT = 10000
NUM_REL = 50
R = 2 * NUM_REL
DIM = 128
NB = 4
E = 320000

_NTILES = 32
_EPT = E // _NTILES
_KE = 40
_PT = 62720
_NRPAD = 16 * _PT
_ZB = 12544
_KP = 160
_KC = 80
_ROWS_PER_TILE = N_ENT // 16


def _f32(shape):
    return jax.ShapeDtypeStruct(shape, jnp.float32)


@functools.partial(
    pl.kernel,
    out_type=(_f32((E * 4,)), _f32((E * 4,))),
    mesh=plsc.VectorSubcoreMesh(core_axis_name="c", subcore_axis_name="s"),
    scratch_types=[
        pltpu.VMEM_SHARED((_NRPAD,), jnp.float32),
        pltpu.VMEM((_ZB,), jnp.float32),
        pltpu.VMEM((_KP,), jnp.int32),
        pltpu.VMEM((_KP,), jnp.int32),
        pltpu.VMEM((_KP,), jnp.int32),
        pltpu.VMEM((_KP,), jnp.float32),
        pltpu.VMEM((_KC * 4,), jnp.int32),
        pltpu.VMEM((_KC * 4,), jnp.int32),
        pltpu.VMEM((_KC * 4,), jnp.float32),
        pltpu.VMEM((_KC * 4,), jnp.float32),
        pltpu.VMEM((_KC * 4,), jnp.float32),
        pltpu.VMEM((_KC * 4,), jnp.float32),
        pltpu.VMEM((_KC * 4,), jnp.float32),
        pltpu.SemaphoreType.DMA,
    ],
)
def _prep(dst_hbm, rel_hbm, a0_hbm, a1_hbm, c0_hbm, c1_hbm,
          counts, zbuf, dst_v, rel_v, key_v, ones_v, key4_v, idx4_v,
          a0g_v, a1g_v, cnt4_v, cb0, cb1, sem):
    cid = lax.axis_index("c")
    sid = lax.axis_index("s")

    def fill(i, _):
        zbuf[pl.ds(i * 16, 16)] = jnp.zeros((16,), jnp.float32)
        ov = i * 16
        ones_v[pl.ds(lax.rem(ov, _KP), 16)] = jnp.full((16,), 1.0, jnp.float32)
        return 0
    lax.fori_loop(0, _ZB // 16, fill, 0)
    for k in range(_PT // _ZB):
        pltpu.sync_copy(zbuf, counts.at[pl.ds(sid * _PT + k * _ZB, _ZB)])
    plsc.subcore_barrier()

    tbase = sid * (E // 16)

    def count_chunk(ci, _):
        base = tbase + ci * _KP
        pltpu.sync_copy(dst_hbm.at[pl.ds(base, _KP)], dst_v)
        pltpu.sync_copy(rel_hbm.at[pl.ds(base, _KP)], rel_v)

        def keys(j, _):
            key_v[pl.ds(j * 16, 16)] = (
                dst_v[pl.ds(j * 16, 16)] * R + rel_v[pl.ds(j * 16, 16)])
            return 0
        lax.fori_loop(0, _KP // 16, keys, 0)
        pltpu.sync_copy(ones_v, counts.at[key_v], add=True)
        return 0
    lax.fori_loop(0, (E // 16) // _KP, count_chunk, 0)
    plsc.subcore_barrier()

    iot = lax.iota(jnp.int32, 16)
    tb3 = cid * (E // 2) + sid * (E // 32)

    lane_b = lax.rem(iot, 4)

    def c_chunk(ci, _):
        base = tb3 + ci * _KC
        pltpu.sync_copy(dst_hbm.at[pl.ds(base, _KC)], dst_v.at[pl.ds(0, _KC)])
        pltpu.sync_copy(rel_hbm.at[pl.ds(base, _KC)], rel_v.at[pl.ds(0, _KC)])

        def expand(q, _):
            dv = dst_v[pl.ds(q * 16, 16)]
            tv = rel_v[pl.ds(q * 16, 16)]
            kv = dv * R + tv
            for m in range(4):
                k0, k1 = kv[4 * m], kv[4 * m + 1]
                k2, k3 = kv[4 * m + 2], kv[4 * m + 3]
                t0, t1 = tv[4 * m], tv[4 * m + 1]
                t2, t3 = tv[4 * m + 2], tv[4 * m + 3]
                krep = jnp.where(
                    iot < 4, k0,
                    jnp.where(iot < 8, k1, jnp.where(iot < 12, k2, k3)))
                trep = jnp.where(
                    iot < 4, t0,
                    jnp.where(iot < 8, t1, jnp.where(iot < 12, t2, t3)))
                off = (q * 4 + m) * 16
                key4_v[pl.ds(off, 16)] = krep
                idx4_v[pl.ds(off, 16)] = trep * 4 + lane_b
            return 0
        lax.fori_loop(0, _KC // 16, expand, 0)
        pltpu.async_copy(counts.at[key4_v], cnt4_v, sem).wait()
        pltpu.async_copy(a0_hbm.at[idx4_v], a0g_v, sem).wait()
        pltpu.async_copy(a1_hbm.at[idx4_v], a1g_v, sem).wait()

        def cgrp(q, _):
            nrm4 = 1.0 / jnp.maximum(cnt4_v[pl.ds(q * 16, 16)], 1.0)
            cb0[pl.ds(q * 16, 16)] = a0g_v[pl.ds(q * 16, 16)] * nrm4
            cb1[pl.ds(q * 16, 16)] = a1g_v[pl.ds(q * 16, 16)] * nrm4
            return 0
        lax.fori_loop(0, (_KC * 4) // 16, cgrp, 0)
        pltpu.sync_copy(cb0, c0_hbm.at[pl.ds(base * 4, _KC * 4)])
        pltpu.sync_copy(cb1, c1_hbm.at[pl.ds(base * 4, _KC * 4)])
        return 0
    lax.fori_loop(0, (E // 32) // _KC, c_chunk, 0)


@functools.partial(
    pl.kernel,
    out_type=_f32((2, N_ENT, DIM)),
    mesh=plsc.VectorSubcoreMesh(core_axis_name="c", subcore_axis_name="s"),
    scratch_types=[
        pltpu.VMEM_SHARED((N_ENT, DIM), jnp.float32),
        pltpu.VMEM((_KE,), jnp.int32),
        pltpu.VMEM((_KE,), jnp.int32),
        pltpu.VMEM((_KE * 4,), jnp.float32),
        pltpu.VMEM((_KE, NB * DIM), jnp.float32),
        pltpu.VMEM((_KE, DIM), jnp.float32),
        pltpu.VMEM((104, DIM), jnp.float32),
        pltpu.SemaphoreType.DMA,
    ],
)
def _edge_pass(xt_hbm, src_hbm, dst_hbm, c_hbm, out_hbm,
               acc, src_v, dst_v, c_v, rows_v, msg_v, zb, sem):
    cid = lax.axis_index("c")
    sid = lax.axis_index("s")

    def zf(i, _):
        for g in range(DIM // 16):
            zb[i, pl.ds(g * 16, 16)] = jnp.zeros((16,), jnp.float32)
        return 0
    lax.fori_loop(0, 104, zf, 0)
    for k in range(6):
        pltpu.sync_copy(zb, acc.at[pl.ds(sid * 624 + k * 104, 104)])

    @pl.when(sid == 15)
    def _():
        pltpu.sync_copy(zb.at[pl.ds(0, 16)], acc.at[pl.ds(9984, 16)])
    plsc.subcore_barrier()

    ebase = cid * (E // 2) + sid * _EPT

    def chunk(ci, _):
        base = ebase + ci * _KE
        pltpu.sync_copy(src_hbm.at[pl.ds(base, _KE)], src_v)
        pltpu.sync_copy(dst_hbm.at[pl.ds(base, _KE)], dst_v)
        pltpu.sync_copy(c_hbm.at[pl.ds(base * 4, _KE * 4)], c_v)
        pltpu.async_copy(xt_hbm.at[src_v], rows_v, sem).wait()

        def edge4(q, _):
            cv = c_v[pl.ds(q * 16, 16)]
            for m in range(4):
                e = q * 4 + m
                c0 = cv[4 * m]
                c1 = cv[4 * m + 1]
                c2 = cv[4 * m + 2]
                c3 = cv[4 * m + 3]
                for g in range(DIM // 16):
                    msg_v[e, pl.ds(g * 16, 16)] = (
                        c0 * rows_v[e, pl.ds(g * 16, 16)]
                        + c1 * rows_v[e, pl.ds(DIM + g * 16, 16)]
                        + c2 * rows_v[e, pl.ds(2 * DIM + g * 16, 16)]
                        + c3 * rows_v[e, pl.ds(3 * DIM + g * 16, 16)])
            return 0
        lax.fori_loop(0, _KE // 4, edge4, 0)
        pltpu.sync_copy(msg_v, acc.at[dst_v], add=True)
        return 0
    lax.fori_loop(0, _EPT // _KE, chunk, 0)
    plsc.subcore_barrier()
    for k in range(6):
        r0 = sid * 624 + k * 104
        pltpu.sync_copy(acc.at[pl.ds(r0, 104)], out_hbm.at[cid, pl.ds(r0, 104)])

    @pl.when(sid == 15)
    def _():
        pltpu.sync_copy(acc.at[pl.ds(9984, 16)], out_hbm.at[cid, pl.ds(9984, 16)])


def _tc_in_body(a_ref, w_ref, b_ref, vc_ref, rw_ref, rb_ref, xt_ref, rt_ref):
    x = jnp.dot(a_ref[...], w_ref[...],
                preferred_element_type=jnp.float32) + b_ref[...]
    xt_ref[...] = jnp.dot(x, vc_ref[...], preferred_element_type=jnp.float32)
    rt_ref[...] = jnp.dot(x, rw_ref[...],
                          preferred_element_type=jnp.float32) + rb_ref[...]


def _tc_mid_body(p_ref, rt0_ref, vc_ref, rw_ref, rb_ref, xt_ref, rt_ref):
    x = jnp.tanh(p_ref[0] + p_ref[1] + rt0_ref[...])
    xt_ref[...] = jnp.dot(x, vc_ref[...], preferred_element_type=jnp.float32)
    rt_ref[...] = jnp.dot(x, rw_ref[...],
                          preferred_element_type=jnp.float32) + rb_ref[...]


def _tc_fin_body(p_ref, rt_ref, o_ref):
    o_ref[...] = p_ref[0] + p_ref[1] + rt_ref[...]


def _tc_rel_body(a_ref, w_ref, o_ref):
    o_ref[...] = jnp.dot(a_ref[...], w_ref[...],
                         preferred_element_type=jnp.float32)


_BN = 400
_GRID = N_ENT // _BN

_full128 = pl.BlockSpec((DIM, DIM), lambda i: (0, 0))
_full512 = pl.BlockSpec((DIM, NB * DIM), lambda i: (0, 0))
_bias = pl.BlockSpec((1, DIM), lambda i: (0, 0))
_rows128 = pl.BlockSpec((_BN, DIM), lambda i: (i, 0))
_rows512 = pl.BlockSpec((_BN, NB * DIM), lambda i: (i, 0))
_prow = pl.BlockSpec((2, _BN, DIM), lambda i: (0, i, 0))


def _tc_in(x0, W_in, b_in, vcat, root, bias):
    return pl.pallas_call(
        _tc_in_body,
        grid=(_GRID,),
        in_specs=[_rows128, _full128, _bias, _full512, _full128, _bias],
        out_specs=[_rows512, _rows128],
        out_shape=[_f32((N_ENT, NB * DIM)), _f32((N_ENT, DIM))],
    )(x0, W_in, b_in, vcat, root, bias)


def _tc_mid(pout, rt0, vcat, root, bias):
    return pl.pallas_call(
        _tc_mid_body,
        grid=(_GRID,),
        in_specs=[_prow, _rows128, _full512, _full128, _bias],
        out_specs=[_rows512, _rows128],
        out_shape=[_f32((N_ENT, NB * DIM)), _f32((N_ENT, DIM))],
    )(pout, rt0, vcat, root, bias)


def _tc_fin(pout, rt1):
    return pl.pallas_call(
        _tc_fin_body,
        grid=(_GRID,),
        in_specs=[_prow, _rows128],
        out_specs=_rows128,
        out_shape=_f32((N_ENT, DIM)),
    )(pout, rt1)


def _tc_rel(init_rel, w_rel):
    return pl.pallas_call(
        _tc_rel_body,
        in_specs=[pl.BlockSpec((R, DIM), lambda: (0, 0)),
                  pl.BlockSpec((DIM, DIM), lambda: (0, 0))],
        out_specs=pl.BlockSpec((R, DIM), lambda: (0, 0)),
        out_shape=_f32((R, DIM)),
    )(init_rel, w_rel)


def kernel(init_embed, init_rel, w_rel, W_in, b_in, V0, a0, root0, bias0,
           V1, a1, root1, bias1, edge_index, edge_type):
    src = edge_index[0].astype(jnp.int32)
    dst = edge_index[1].astype(jnp.int32)
    rel = edge_type.astype(jnp.int32)
    vcat0 = V0.transpose(1, 0, 2).reshape(DIM, NB * DIM)
    vcat1 = V1.transpose(1, 0, 2).reshape(DIM, NB * DIM)

    c0, c1 = _prep(dst, rel, a0.reshape(-1), a1.reshape(-1))

    xt0, rt0 = _tc_in(init_embed, W_in, b_in.reshape(1, DIM), vcat0, root0,
                      bias0.reshape(1, DIM))
    p0 = _edge_pass(xt0, src, dst, c0)
    xt1, rt1 = _tc_mid(p0, rt0, vcat1, root1, bias1.reshape(1, DIM))
    p1 = _edge_pass(xt1, src, dst, c1)
    x_out = _tc_fin(p1, rt1)
    r_out = _tc_rel(init_rel, w_rel)
    return (x_out, r_out)
